# Initial kernel scaffold; baseline (speedup 1.0000x reference)
import jax
import jax.numpy as jnp
from jax.experimental import pallas as pl


def kernel(input_ids, embedding_table, hard_norm_weight, proj_weight):
    raise NotImplementedError("write your pallas kernel here")


# speedup vs baseline: 2.8105x; 2.8105x over previous
import functools

import jax
import jax.numpy as jnp
from jax import lax
from jax.experimental import pallas as pl
from jax.experimental.pallas import tpu as pltpu
from jax.experimental.pallas import tpu_sc as plsc

VOCAB = 128
MM_HIDDEN = 2048
TXT_HIDDEN = 2048
EPS = 1e-6

NC, NS = 2, 16
NW = NC * NS
TOKENS = 4 * 2048
B_PER_W = TOKENS // NW
CHUNK = 32
NCHUNK = B_PER_W // CHUNK


def _lut_body(table_ref, w_ref, proj_ref, out_ref):
    x = table_ref[...]
    var = jnp.mean(x * x, axis=-1, keepdims=True)
    normed = x * lax.rsqrt(var + EPS) * (1.0 + w_ref[...])
    y = lax.dot_general(
        normed, proj_ref[...],
        dimension_numbers=(((1,), (1,)), ((), ())),
        preferred_element_type=jnp.float32,
    )
    var2 = jnp.mean(y * y, axis=-1, keepdims=True)
    out_ref[...] = y * lax.rsqrt(var2 + EPS)


def _compute_lut(embedding_table, hard_norm_weight, proj_weight):
    return pl.pallas_call(
        _lut_body,
        out_shape=jax.ShapeDtypeStruct((VOCAB, TXT_HIDDEN), jnp.float32),
    )(embedding_table, hard_norm_weight.reshape(1, MM_HIDDEN), proj_weight)


def _gather_body(lut_hbm, ids_hbm, out_hbm, idx_v, rows_v, sem):
    wid = lax.axis_index("s") * NC + lax.axis_index("c")
    pltpu.sync_copy(ids_hbm.at[wid], idx_v)
    for c in range(NCHUNK):
        pltpu.async_copy(lut_hbm.at[idx_v.at[c]], rows_v, sem).wait()
        base = (wid * NCHUNK + c) * CHUNK
        pltpu.sync_copy(rows_v, out_hbm.at[pl.ds(base, CHUNK)])


@functools.lru_cache(maxsize=1)
def _build_gather():
    return pl.kernel(
        _gather_body,
        out_type=jax.ShapeDtypeStruct((TOKENS, TXT_HIDDEN), jnp.float32),
        mesh=plsc.VectorSubcoreMesh(core_axis_name="c", subcore_axis_name="s"),
        scratch_types=[
            pltpu.VMEM((NCHUNK, CHUNK), jnp.int32),
            pltpu.VMEM((CHUNK, TXT_HIDDEN), jnp.float32),
            pltpu.SemaphoreType.DMA,
        ],
    )


def kernel(input_ids, embedding_table, hard_norm_weight, proj_weight):
    lut = _compute_lut(embedding_table, hard_norm_weight, proj_weight)
    ids3 = input_ids.reshape(NW, NCHUNK, CHUNK)
    out = _build_gather()(lut, ids3)
    return out.reshape(input_ids.shape[0], input_ids.shape[1], TXT_HIDDEN)
